# Initial kernel scaffold; baseline (speedup 1.0000x reference)
import jax
import jax.numpy as jnp
from jax.experimental import pallas as pl


def kernel(loc_data, conf_data, landm_data, priors, targets):
    raise NotImplementedError("write your pallas kernel here")


# speedup vs baseline: 50.8106x; 50.8106x over previous
import jax
import jax.numpy as jnp
from jax.experimental import pallas as pl

P = 16800
NUM_CLASSES = 2
THRESHOLD = 0.35
NEGPOS_RATIO = 7
VAR0, VAR1 = 0.1, 0.2
LANES = 128
ROWS = (P + LANES - 1) // LANES
PADP = ROWS * LANES - P


def _body(loc_ref, conf_ref, landm_ref, pri_ref, tgt_ref, out_ref):
    f32 = jnp.float32
    row = jax.lax.broadcasted_iota(jnp.int32, (ROWS, LANES), 0)
    lane = jax.lax.broadcasted_iota(jnp.int32, (ROWS, LANES), 1)
    flat_idx = row * LANES + lane
    valid = flat_idx < P

    px = pri_ref[0]
    py = pri_ref[1]
    pw = pri_ref[2]
    ph = pri_ref[3]
    pf_x1 = px - pw * 0.5
    pf_y1 = py - ph * 0.5
    pf_x2 = px + pw * 0.5
    pf_y2 = py + ph * 0.5
    area_p = (pf_x2 - pf_x1) * (pf_y2 - pf_y1)

    bto = jnp.full((ROWS, LANES), -1.0, f32)
    bti = jnp.zeros((ROWS, LANES), jnp.int32)
    m_list = []
    bpi_list = []
    BIG = jnp.int32(P + 2)
    for i in range(8):
        tx1 = tgt_ref[0, i, 0]
        ty1 = tgt_ref[0, i, 1]
        tx2 = tgt_ref[0, i, 2]
        ty2 = tgt_ref[0, i, 3]
        iw = jnp.maximum(jnp.minimum(pf_x2, tx2) - jnp.maximum(pf_x1, tx1), 0.0)
        ih = jnp.maximum(jnp.minimum(pf_y2, ty2) - jnp.maximum(pf_y1, ty1), 0.0)
        inter = iw * ih
        at = (tx2 - tx1) * (ty2 - ty1)
        iou = inter / (at + area_p - inter)
        iou = jnp.where(valid, iou, -1.0)
        upd = iou > bto
        bti = jnp.where(upd, i, bti)
        bto = jnp.where(upd, iou, bto)
        m_i = jnp.max(iou)
        bpi_i = jnp.min(jnp.where(iou == m_i, flat_idx, BIG))
        m_list.append(m_i)
        bpi_list.append(bpi_i)

    bto0 = bto
    for i in range(8):
        valid_gt = m_list[i] >= 0.2
        cur = jnp.sum(jnp.where(flat_idx == bpi_list[i], bto0, 0.0))
        fill = jnp.where(valid_gt, 2.0, cur)
        hit = flat_idx == bpi_list[i]
        bto = jnp.where(hit, fill, bto)
        bti = jnp.where(hit, i, bti)

    oh = [(bti == i).astype(f32) for i in range(8)]
    conf_raw = jnp.zeros((ROWS, LANES), f32)
    for i in range(8):
        conf_raw = conf_raw + oh[i] * tgt_ref[0, i, 14]
    conf_t = jnp.where(bto < THRESHOLD, 0.0, conf_raw)
    conf_t = jnp.where(valid, conf_t, 0.0)
    pos = conf_t != 0.0
    pos1 = conf_t > 0.0
    posf = pos.astype(f32)
    pos1f = pos1.astype(f32)
    num_pos = jnp.sum(posf)
    num_pos_landm = jnp.sum(pos1f)

    def gather8(col):
        acc = jnp.zeros((ROWS, LANES), f32)
        for i in range(8):
            acc = acc + oh[i] * tgt_ref[0, i, col]
        return acc

    def sl1(dv):
        a = jnp.abs(dv)
        return jnp.where(a < 1.0, 0.5 * dv * dv, a - 0.5)

    mx1 = gather8(0)
    my1 = gather8(1)
    mx2 = gather8(2)
    my2 = gather8(3)
    g_cx = ((mx1 + mx2) * 0.5 - px) / (VAR0 * pw)
    g_cy = ((my1 + my2) * 0.5 - py) / (VAR0 * ph)
    g_w = jnp.log((mx2 - mx1) / pw) / VAR1
    g_h = jnp.log((my2 - my1) / ph) / VAR1
    loss_l = jnp.sum(
        (sl1(loc_ref[0, 0] - g_cx) + sl1(loc_ref[0, 1] - g_cy)
         + sl1(loc_ref[0, 2] - g_w) + sl1(loc_ref[0, 3] - g_h)) * posf)

    lacc = jnp.zeros((ROWS, LANES), f32)
    for j in range(10):
        mlm = gather8(4 + j)
        pc = px if j % 2 == 0 else py
        ps = pw if j % 2 == 0 else ph
        g = (mlm - pc) / (VAR0 * ps)
        lacc = lacc + sl1(landm_ref[0, j] - g)
    loss_landm = jnp.sum(lacc * pos1f)

    c0 = conf_ref[0, 0]
    c1 = conf_ref[0, 1]
    d = c1 - c0
    lse = jnp.maximum(c0, c1) + jnp.log(1.0 + jnp.exp(-jnp.abs(d)))
    ce_pos = lse - c1
    ce_neg = lse - c0
    pos_sum = jnp.sum(ce_pos * posf)

    K = jnp.minimum(jnp.int32(NEGPOS_RATIO) * jnp.sum(pos.astype(jnp.int32)),
                    jnp.int32(P - 1))
    SENT = jnp.int32(-2147483648)
    bits = jax.lax.bitcast_convert_type(d, jnp.int32)
    s = jnp.where(bits >= 0, bits, SENT - bits - 1)
    s = jnp.where(pos | jnp.logical_not(valid), SENT, s)

    def bisect_body(_, carry):
        lo, hi = carry
        mid = (lo >> 1) + (hi >> 1) + ((lo | hi) & jnp.int32(1))
        cnt = jnp.sum((s >= mid).astype(jnp.int32))
        take = cnt >= K
        return (jnp.where(take, mid, lo), jnp.where(take, hi, mid - jnp.int32(1)))

    lo, _ = jax.lax.fori_loop(0, 32, bisect_body,
                              (SENT, jnp.int32(2147483647)))
    T = lo
    gt = (s > T).astype(f32)
    eq = (s == T).astype(f32)
    c_gt = jnp.sum(gt)
    c_eq = jnp.sum(eq)
    k_rem = jnp.where(T > SENT, K.astype(f32) - c_gt, 0.0)
    neg_sum = jnp.sum(ce_neg * gt) + \
        k_rem / jnp.maximum(c_eq, 1.0) * jnp.sum(ce_neg * eq)
    loss_c = pos_sum + neg_sum

    lane_row = jax.lax.broadcasted_iota(jnp.int32, (1, LANES), 1)
    out = (jnp.where(lane_row == 0, loss_l, 0.0)
           + jnp.where(lane_row == 1, loss_c, 0.0)
           + jnp.where(lane_row == 2, loss_landm, 0.0)
           + jnp.where(lane_row == 3, num_pos, 0.0)
           + jnp.where(lane_row == 4, num_pos_landm, 0.0))
    out_ref[...] = out.reshape(1, 1, LANES)


def kernel(loc_data, conf_data, landm_data, priors, targets):
    B = loc_data.shape[0]
    targets = jax.lax.stop_gradient(targets)
    priors = jax.lax.stop_gradient(priors)

    def prep(x):
        xt = jnp.transpose(x, (0, 2, 1))
        xt = jnp.pad(xt, ((0, 0), (0, 0), (0, PADP)))
        return xt.reshape(B, x.shape[2], ROWS, LANES)

    locT = prep(loc_data)
    confT = prep(conf_data)
    landmT = prep(landm_data)
    priT = jnp.pad(jnp.transpose(priors), ((0, 0), (0, PADP)),
                   constant_values=1.0).reshape(4, ROWS, LANES)

    o = pl.pallas_call(
        _body,
        grid=(B,),
        in_specs=[
            pl.BlockSpec((1, 4, ROWS, LANES), lambda b: (b, 0, 0, 0)),
            pl.BlockSpec((1, NUM_CLASSES, ROWS, LANES), lambda b: (b, 0, 0, 0)),
            pl.BlockSpec((1, 10, ROWS, LANES), lambda b: (b, 0, 0, 0)),
            pl.BlockSpec((4, ROWS, LANES), lambda b: (0, 0, 0)),
            pl.BlockSpec((1, 8, 15), lambda b: (b, 0, 0)),
        ],
        out_specs=pl.BlockSpec((1, 1, LANES), lambda b: (b, 0, 0)),
        out_shape=jax.ShapeDtypeStruct((B, 1, LANES), jnp.float32),
    )(locT, confT, landmT, priT, targets)

    N = jnp.maximum(jnp.sum(o[:, 0, 3]), 1.0)
    N1 = jnp.maximum(jnp.sum(o[:, 0, 4]), 1.0)
    return (jnp.sum(o[:, 0, 0]) / N, jnp.sum(o[:, 0, 1]) / N,
            jnp.sum(o[:, 0, 2]) / N1)


# speedup vs baseline: 60.3861x; 1.1885x over previous
import dataclasses

import jax
import jax.numpy as jnp
from jax.experimental import pallas as pl
from jax.experimental.pallas import tpu as pltpu
from jax.experimental.pallas import tpu_sc as plsc

P = 16800
NUM_CLASSES = 2
THRESHOLD = 0.35
NEGPOS_RATIO = 7
VAR0, VAR1 = 0.1, 0.2
LANES = 128
ROWS = (P + LANES - 1) // LANES
PADP = ROWS * LANES - P
NV = P // 16
NSUB = 16


def _sc_match_body(pri_hbm, tgt_hbm, conf_out, bti_out, pvec, tsm, bto_ref, bti_ref):
    f32 = jnp.float32
    c = jax.lax.axis_index("c")
    s = jax.lax.axis_index("s")
    b = c * NSUB + s
    pltpu.sync_copy(pri_hbm, pvec)
    pltpu.sync_copy(tgt_hbm.at[b], tsm)

    lane = jax.lax.iota(jnp.int32, 16)
    BIG = jnp.int32(P + 2)

    trow = [tsm[pl.ds(16 * i, 16)] for i in range(8)]
    tx1 = [trow[i][0] for i in range(8)]
    ty1 = [trow[i][1] for i in range(8)]
    tx2 = [trow[i][2] for i in range(8)]
    ty2 = [trow[i][3] for i in range(8)]
    lab = [trow[i][14] for i in range(8)]
    at = [(tx2[i] - tx1[i]) * (ty2[i] - ty1[i]) for i in range(8)]

    init = tuple([jnp.full((16,), -2.0, f32) for _ in range(8)]
                 + [jnp.zeros((16,), jnp.int32) for _ in range(8)])

    def body(i, carry):
        rm = list(carry[0:8])
        ra = list(carry[8:16])
        off = i * 16
        gidx = off + lane
        px = pvec[pl.ds(off, 16)]
        py = pvec[pl.ds(P + off, 16)]
        pw = pvec[pl.ds(2 * P + off, 16)]
        ph = pvec[pl.ds(3 * P + off, 16)]
        pf_x1 = px - pw * 0.5
        pf_y1 = py - ph * 0.5
        pf_x2 = px + pw * 0.5
        pf_y2 = py + ph * 0.5
        area_p = (pf_x2 - pf_x1) * (pf_y2 - pf_y1)
        bto_v = jnp.full((16,), -1.0, f32)
        bti_v = jnp.zeros((16,), jnp.int32)
        for i_t in range(8):
            iw = jnp.maximum(
                jnp.minimum(pf_x2, tx2[i_t]) - jnp.maximum(pf_x1, tx1[i_t]), 0.0)
            ih = jnp.maximum(
                jnp.minimum(pf_y2, ty2[i_t]) - jnp.maximum(pf_y1, ty1[i_t]), 0.0)
            inter = iw * ih
            iou = inter / (at[i_t] + area_p - inter)
            upd = iou > bto_v
            bti_v = jnp.where(upd, i_t, bti_v)
            bto_v = jnp.where(upd, iou, bto_v)
            upd2 = iou > rm[i_t]
            ra[i_t] = jnp.where(upd2, gidx, ra[i_t])
            rm[i_t] = jnp.where(upd2, iou, rm[i_t])
        bto_ref[pl.ds(off, 16)] = bto_v
        bti_ref[pl.ds(off, 16)] = bti_v
        return tuple(rm + ra)

    carry = jax.lax.fori_loop(0, NV, body, init)
    rm = carry[0:8]
    ra = carry[8:16]

    m = [jnp.max(rm[i]) for i in range(8)]
    bpi = [jnp.min(jnp.where(rm[i] == m[i], ra[i], BIG)) for i in range(8)]
    base = [jnp.bitwise_and(bpi[i], jnp.int32(~15)) for i in range(8)]
    hit = [(base[i] + lane) == bpi[i] for i in range(8)]
    orig = [jnp.sum(jnp.where(hit[i], bto_ref[pl.ds(base[i], 16)], 0.0))
            for i in range(8)]
    for i in range(8):
        fill = jnp.where(m[i] >= 0.2, 2.0, orig[i])
        sl = bto_ref[pl.ds(base[i], 16)]
        bto_ref[pl.ds(base[i], 16)] = jnp.where(hit[i], fill, sl)
        sli = bti_ref[pl.ds(base[i], 16)]
        bti_ref[pl.ds(base[i], 16)] = jnp.where(hit[i], jnp.int32(i), sli)

    def conf_body(i, _):
        off = i * 16
        bto_v = bto_ref[pl.ds(off, 16)]
        bti_v = bti_ref[pl.ds(off, 16)]
        cv = jnp.zeros((16,), f32)
        for i_t in range(8):
            cv = jnp.where(bti_v == i_t, lab[i_t], cv)
        cv = jnp.where(bto_v < THRESHOLD, 0.0, cv)
        bto_ref[pl.ds(off, 16)] = cv
        return 0

    jax.lax.fori_loop(0, NV, conf_body, 0)
    pltpu.sync_copy(bto_ref, conf_out.at[b])
    pltpu.sync_copy(bti_ref, bti_out.at[b])


def _tc_body(loc_ref, conf_ref, landm_ref, pri_ref, tgt_ref, ct_ref, bi_ref,
             out_ref):
    f32 = jnp.float32
    row = jax.lax.broadcasted_iota(jnp.int32, (ROWS, LANES), 0)
    lane = jax.lax.broadcasted_iota(jnp.int32, (ROWS, LANES), 1)
    flat_idx = row * LANES + lane
    valid = flat_idx < P

    px = pri_ref[0]
    py = pri_ref[1]
    pw = pri_ref[2]
    ph = pri_ref[3]

    conf_t = jnp.where(valid, ct_ref[0], 0.0)
    bti = bi_ref[0]
    pos = conf_t != 0.0
    pos1 = conf_t > 0.0
    posf = pos.astype(f32)
    pos1f = pos1.astype(f32)
    num_pos = jnp.sum(posf)
    num_pos_landm = jnp.sum(pos1f)

    oh = [(bti == i).astype(f32) for i in range(8)]

    def gather8(col):
        acc = jnp.zeros((ROWS, LANES), f32)
        for i in range(8):
            acc = acc + oh[i] * tgt_ref[0, i, col]
        return acc

    def sl1(dv):
        a = jnp.abs(dv)
        return jnp.where(a < 1.0, 0.5 * dv * dv, a - 0.5)

    mx1 = gather8(0)
    my1 = gather8(1)
    mx2 = gather8(2)
    my2 = gather8(3)
    g_cx = ((mx1 + mx2) * 0.5 - px) / (VAR0 * pw)
    g_cy = ((my1 + my2) * 0.5 - py) / (VAR0 * ph)
    g_w = jnp.log((mx2 - mx1) / pw) / VAR1
    g_h = jnp.log((my2 - my1) / ph) / VAR1
    loss_l = jnp.sum(
        (sl1(loc_ref[0, 0] - g_cx) + sl1(loc_ref[0, 1] - g_cy)
         + sl1(loc_ref[0, 2] - g_w) + sl1(loc_ref[0, 3] - g_h)) * posf)

    lacc = jnp.zeros((ROWS, LANES), f32)
    for j in range(10):
        mlm = gather8(4 + j)
        pc = px if j % 2 == 0 else py
        ps = pw if j % 2 == 0 else ph
        g = (mlm - pc) / (VAR0 * ps)
        lacc = lacc + sl1(landm_ref[0, j] - g)
    loss_landm = jnp.sum(lacc * pos1f)

    c0 = conf_ref[0, 0]
    c1 = conf_ref[0, 1]
    d = c1 - c0
    lse = jnp.maximum(c0, c1) + jnp.log(1.0 + jnp.exp(-jnp.abs(d)))
    ce_pos = lse - c1
    ce_neg = lse - c0
    pos_sum = jnp.sum(ce_pos * posf)

    K = jnp.minimum(jnp.int32(NEGPOS_RATIO) * jnp.sum(pos.astype(jnp.int32)),
                    jnp.int32(P - 1))
    SENT = jnp.int32(-2147483648)
    bits = jax.lax.bitcast_convert_type(d, jnp.int32)
    s = jnp.where(bits >= 0, bits, SENT - bits - 1)
    s = jnp.where(pos | jnp.logical_not(valid), SENT, s)

    def bisect_body(_, carry):
        lo, hi = carry
        mid = (lo >> 1) + (hi >> 1) + ((lo | hi) & jnp.int32(1))
        cnt = jnp.sum((s >= mid).astype(jnp.int32))
        take = cnt >= K
        return (jnp.where(take, mid, lo), jnp.where(take, hi, mid - jnp.int32(1)))

    lo, _ = jax.lax.fori_loop(0, 32, bisect_body,
                              (SENT, jnp.int32(2147483647)))
    T = lo
    gt = (s > T).astype(f32)
    eq = (s == T).astype(f32)
    c_gt = jnp.sum(gt)
    c_eq = jnp.sum(eq)
    k_rem = jnp.where(T > SENT, K.astype(f32) - c_gt, 0.0)
    neg_sum = jnp.sum(ce_neg * gt) + \
        k_rem / jnp.maximum(c_eq, 1.0) * jnp.sum(ce_neg * eq)
    loss_c = pos_sum + neg_sum

    lane_row = jax.lax.broadcasted_iota(jnp.int32, (1, LANES), 1)
    out = (jnp.where(lane_row == 0, loss_l, 0.0)
           + jnp.where(lane_row == 1, loss_c, 0.0)
           + jnp.where(lane_row == 2, loss_landm, 0.0)
           + jnp.where(lane_row == 3, num_pos, 0.0)
           + jnp.where(lane_row == 4, num_pos_landm, 0.0))
    out_ref[...] = out.reshape(1, 1, LANES)


def kernel(loc_data, conf_data, landm_data, priors, targets):
    B = loc_data.shape[0]
    targets = jax.lax.stop_gradient(targets)
    priors = jax.lax.stop_gradient(priors)

    pri_flat = jnp.transpose(priors).reshape(4 * P)
    tgt_pad = jnp.pad(targets, ((0, 0), (0, 0), (0, 1))).reshape(B, 128)
    mesh = plsc.VectorSubcoreMesh(core_axis_name="c", subcore_axis_name="s")
    cp = pltpu.CompilerParams()
    if "needs_layout_passes" in pltpu.CompilerParams.__dataclass_fields__:
        cp = dataclasses.replace(cp, needs_layout_passes=False)
    conf_t, btiA = pl.kernel(
        _sc_match_body,
        out_type=(jax.ShapeDtypeStruct((B, P), jnp.float32),
                  jax.ShapeDtypeStruct((B, P), jnp.int32)),
        mesh=mesh,
        scratch_types=[pltpu.VMEM((4 * P,), jnp.float32),
                       pltpu.VMEM((128,), jnp.float32),
                       pltpu.VMEM((P,), jnp.float32),
                       pltpu.VMEM((P,), jnp.int32)],
        compiler_params=cp,
    )(pri_flat, tgt_pad)

    def prep(x):
        xt = jnp.transpose(x, (0, 2, 1))
        xt = jnp.pad(xt, ((0, 0), (0, 0), (0, PADP)))
        return xt.reshape(B, x.shape[2], ROWS, LANES)

    locT = prep(loc_data)
    confT = prep(conf_data)
    landmT = prep(landm_data)
    priT = jnp.pad(jnp.transpose(priors), ((0, 0), (0, PADP)),
                   constant_values=1.0).reshape(4, ROWS, LANES)
    ctP = jnp.pad(conf_t, ((0, 0), (0, PADP))).reshape(B, 1, ROWS, LANES)
    biP = jnp.pad(btiA, ((0, 0), (0, PADP))).reshape(B, 1, ROWS, LANES)

    o = pl.pallas_call(
        _tc_body,
        grid=(B,),
        in_specs=[
            pl.BlockSpec((1, 4, ROWS, LANES), lambda b: (b, 0, 0, 0)),
            pl.BlockSpec((1, NUM_CLASSES, ROWS, LANES), lambda b: (b, 0, 0, 0)),
            pl.BlockSpec((1, 10, ROWS, LANES), lambda b: (b, 0, 0, 0)),
            pl.BlockSpec((4, ROWS, LANES), lambda b: (0, 0, 0)),
            pl.BlockSpec((1, 8, 15), lambda b: (b, 0, 0)),
            pl.BlockSpec((1, 1, ROWS, LANES), lambda b: (b, 0, 0, 0)),
            pl.BlockSpec((1, 1, ROWS, LANES), lambda b: (b, 0, 0, 0)),
        ],
        out_specs=pl.BlockSpec((1, 1, LANES), lambda b: (b, 0, 0)),
        out_shape=jax.ShapeDtypeStruct((B, 1, LANES), jnp.float32),
    )(locT, confT, landmT, priT, targets, ctP, biP)

    N = jnp.maximum(jnp.sum(o[:, 0, 3]), 1.0)
    N1 = jnp.maximum(jnp.sum(o[:, 0, 4]), 1.0)
    return (jnp.sum(o[:, 0, 0]) / N, jnp.sum(o[:, 0, 1]) / N,
            jnp.sum(o[:, 0, 2]) / N1)


# speedup vs baseline: 60.9022x; 1.0085x over previous
import dataclasses

import jax
import jax.numpy as jnp
from jax.experimental import pallas as pl
from jax.experimental.pallas import tpu as pltpu
from jax.experimental.pallas import tpu_sc as plsc

P = 16800
NUM_CLASSES = 2
THRESHOLD = 0.35
NEGPOS_RATIO = 7
VAR0, VAR1 = 0.1, 0.2
LANES = 128
ROWS = (P + LANES - 1) // LANES
PADP = ROWS * LANES - P
NV = P // 16
NSUB = 16


def _sc_match_body(pri_hbm, tgt_hbm, conf_out, bti_out, pvec, tsm, bto_ref, bti_ref):
    f32 = jnp.float32
    c = jax.lax.axis_index("c")
    s = jax.lax.axis_index("s")
    b = c * NSUB + s
    pltpu.sync_copy(pri_hbm, pvec)
    pltpu.sync_copy(tgt_hbm.at[b], tsm)

    lane = jax.lax.iota(jnp.int32, 16)
    BIG = jnp.int32(P + 2)

    trow = [tsm[pl.ds(16 * i, 16)] for i in range(8)]
    tx1 = [trow[i][0] for i in range(8)]
    ty1 = [trow[i][1] for i in range(8)]
    tx2 = [trow[i][2] for i in range(8)]
    ty2 = [trow[i][3] for i in range(8)]
    lab = [trow[i][14] for i in range(8)]
    at = [(tx2[i] - tx1[i]) * (ty2[i] - ty1[i]) for i in range(8)]

    init = tuple([jnp.full((16,), -2.0, f32) for _ in range(8)]
                 + [jnp.zeros((16,), jnp.int32) for _ in range(8)])

    def body(i, carry):
        rm = list(carry[0:8])
        ra = list(carry[8:16])
        off = i * 16
        gidx = off + lane
        px = pvec[pl.ds(off, 16)]
        py = pvec[pl.ds(P + off, 16)]
        pw = pvec[pl.ds(2 * P + off, 16)]
        ph = pvec[pl.ds(3 * P + off, 16)]
        pf_x1 = px - pw * 0.5
        pf_y1 = py - ph * 0.5
        pf_x2 = px + pw * 0.5
        pf_y2 = py + ph * 0.5
        area_p = (pf_x2 - pf_x1) * (pf_y2 - pf_y1)
        bto_v = jnp.full((16,), -1.0, f32)
        bti_v = jnp.zeros((16,), jnp.int32)
        for i_t in range(8):
            iw = jnp.maximum(
                jnp.minimum(pf_x2, tx2[i_t]) - jnp.maximum(pf_x1, tx1[i_t]), 0.0)
            ih = jnp.maximum(
                jnp.minimum(pf_y2, ty2[i_t]) - jnp.maximum(pf_y1, ty1[i_t]), 0.0)
            inter = iw * ih
            iou = inter / (at[i_t] + area_p - inter)
            upd = iou > bto_v
            bti_v = jnp.where(upd, i_t, bti_v)
            bto_v = jnp.where(upd, iou, bto_v)
            upd2 = iou > rm[i_t]
            ra[i_t] = jnp.where(upd2, gidx, ra[i_t])
            rm[i_t] = jnp.where(upd2, iou, rm[i_t])
        bto_ref[pl.ds(off, 16)] = bto_v
        bti_ref[pl.ds(off, 16)] = bti_v
        return tuple(rm + ra)

    carry = jax.lax.fori_loop(0, NV, body, init)
    rm = carry[0:8]
    ra = carry[8:16]

    m = [jnp.max(rm[i]) for i in range(8)]
    bpi = [jnp.min(jnp.where(rm[i] == m[i], ra[i], BIG)) for i in range(8)]
    base = [jnp.bitwise_and(bpi[i], jnp.int32(~15)) for i in range(8)]
    hit = [(base[i] + lane) == bpi[i] for i in range(8)]
    orig = [jnp.sum(jnp.where(hit[i], bto_ref[pl.ds(base[i], 16)], 0.0))
            for i in range(8)]
    for i in range(8):
        fill = jnp.where(m[i] >= 0.2, 2.0, orig[i])
        sl = bto_ref[pl.ds(base[i], 16)]
        bto_ref[pl.ds(base[i], 16)] = jnp.where(hit[i], fill, sl)
        sli = bti_ref[pl.ds(base[i], 16)]
        bti_ref[pl.ds(base[i], 16)] = jnp.where(hit[i], jnp.int32(i), sli)

    def conf_body(i, _):
        off = i * 16
        bto_v = bto_ref[pl.ds(off, 16)]
        bti_v = bti_ref[pl.ds(off, 16)]
        cv = jnp.zeros((16,), f32)
        for i_t in range(8):
            cv = jnp.where(bti_v == i_t, lab[i_t], cv)
        cv = jnp.where(bto_v < THRESHOLD, 0.0, cv)
        bto_ref[pl.ds(off, 16)] = cv
        return 0

    jax.lax.fori_loop(0, NV, conf_body, 0)
    pltpu.sync_copy(bto_ref, conf_out.at[b])
    pltpu.sync_copy(bti_ref, bti_out.at[b])


def _tc_body(loc_ref, conf_ref, landm_ref, pri_ref, tgt_ref, ct_ref, bi_ref,
             out_ref):
    f32 = jnp.float32
    row = jax.lax.broadcasted_iota(jnp.int32, (ROWS, LANES), 0)
    lane = jax.lax.broadcasted_iota(jnp.int32, (ROWS, LANES), 1)
    flat_idx = row * LANES + lane
    valid = flat_idx < P

    px = pri_ref[0]
    py = pri_ref[1]
    inv_vpw = pri_ref[2]
    inv_vph = pri_ref[3]
    log_pw = pri_ref[4]
    log_ph = pri_ref[5]

    conf_t = jnp.where(valid, ct_ref[0], 0.0)
    bti = bi_ref[0]
    pos = conf_t != 0.0
    pos1 = conf_t > 0.0
    posf = pos.astype(f32)
    pos1f = pos1.astype(f32)
    num_pos = jnp.sum(posf)
    num_pos_landm = jnp.sum(pos1f)

    bb0 = (bti & 1) != 0
    bb1 = (bti & 2) != 0
    bb2 = (bti & 4) != 0

    def tree8(vals):
        w = [jnp.where(bb0, vals[2 * j + 1], vals[2 * j]) for j in range(4)]
        x = [jnp.where(bb1, w[2 * k + 1], w[2 * k]) for k in range(2)]
        return jnp.where(bb2, x[1], x[0])

    def sl1(dv):
        a = jnp.abs(dv)
        return jnp.where(a < 1.0, 0.5 * dv * dv, a - 0.5)

    tx1 = [tgt_ref[0, i, 0] for i in range(8)]
    ty1 = [tgt_ref[0, i, 1] for i in range(8)]
    tx2 = [tgt_ref[0, i, 2] for i in range(8)]
    ty2 = [tgt_ref[0, i, 3] for i in range(8)]
    cxm = [(tx1[i] + tx2[i]) * 0.5 for i in range(8)]
    cym = [(ty1[i] + ty2[i]) * 0.5 for i in range(8)]
    logw = [jnp.log(tx2[i] - tx1[i]) for i in range(8)]
    logh = [jnp.log(ty2[i] - ty1[i]) for i in range(8)]

    g_cx = (tree8(cxm) - px) * inv_vpw
    g_cy = (tree8(cym) - py) * inv_vph
    g_w = (tree8(logw) - log_pw) / VAR1
    g_h = (tree8(logh) - log_ph) / VAR1
    loss_l = jnp.sum(
        (sl1(loc_ref[0, 0] - g_cx) + sl1(loc_ref[0, 1] - g_cy)
         + sl1(loc_ref[0, 2] - g_w) + sl1(loc_ref[0, 3] - g_h)) * posf)

    lacc = jnp.zeros((ROWS, LANES), f32)
    for j in range(10):
        mlm = tree8([tgt_ref[0, i, 4 + j] for i in range(8)])
        pc = px if j % 2 == 0 else py
        pi = inv_vpw if j % 2 == 0 else inv_vph
        g = (mlm - pc) * pi
        lacc = lacc + sl1(landm_ref[0, j] - g)
    loss_landm = jnp.sum(lacc * pos1f)

    c0 = conf_ref[0, 0]
    c1 = conf_ref[0, 1]
    d = c1 - c0
    lse = jnp.maximum(c0, c1) + jnp.log(1.0 + jnp.exp(-jnp.abs(d)))
    ce_pos = lse - c1
    ce_neg = lse - c0
    pos_sum = jnp.sum(ce_pos * posf)

    K = jnp.minimum(jnp.int32(NEGPOS_RATIO) * jnp.sum(pos.astype(jnp.int32)),
                    jnp.int32(P - 1))
    SENT = jnp.int32(-2147483648)
    bits = jax.lax.bitcast_convert_type(d, jnp.int32)
    s = jnp.where(bits >= 0, bits, SENT - bits - 1)
    s = jnp.where(pos | jnp.logical_not(valid), SENT, s)

    def bisect_body(_, carry):
        lo, hi = carry
        mid = (lo >> 1) + (hi >> 1) + ((lo | hi) & jnp.int32(1))
        cnt = jnp.sum((s >= mid).astype(jnp.int32))
        take = cnt >= K
        return (jnp.where(take, mid, lo), jnp.where(take, hi, mid - jnp.int32(1)))

    lo, _ = jax.lax.fori_loop(0, 32, bisect_body,
                              (SENT, jnp.int32(2147483647)))
    T = lo
    gt = (s > T).astype(f32)
    eq = (s == T).astype(f32)
    c_gt = jnp.sum(gt)
    c_eq = jnp.sum(eq)
    k_rem = jnp.where(T > SENT, K.astype(f32) - c_gt, 0.0)
    neg_sum = jnp.sum(ce_neg * gt) + \
        k_rem / jnp.maximum(c_eq, 1.0) * jnp.sum(ce_neg * eq)
    loss_c = pos_sum + neg_sum

    lane_row = jax.lax.broadcasted_iota(jnp.int32, (1, LANES), 1)
    out = (jnp.where(lane_row == 0, loss_l, 0.0)
           + jnp.where(lane_row == 1, loss_c, 0.0)
           + jnp.where(lane_row == 2, loss_landm, 0.0)
           + jnp.where(lane_row == 3, num_pos, 0.0)
           + jnp.where(lane_row == 4, num_pos_landm, 0.0))
    out_ref[...] = out.reshape(1, 1, LANES)


def kernel(loc_data, conf_data, landm_data, priors, targets):
    B = loc_data.shape[0]
    targets = jax.lax.stop_gradient(targets)
    priors = jax.lax.stop_gradient(priors)

    pri_flat = jnp.transpose(priors).reshape(4 * P)
    tgt_pad = jnp.pad(targets, ((0, 0), (0, 0), (0, 1))).reshape(B, 128)
    mesh = plsc.VectorSubcoreMesh(core_axis_name="c", subcore_axis_name="s")
    cp = pltpu.CompilerParams()
    if "needs_layout_passes" in pltpu.CompilerParams.__dataclass_fields__:
        cp = dataclasses.replace(cp, needs_layout_passes=False)
    conf_t, btiA = pl.kernel(
        _sc_match_body,
        out_type=(jax.ShapeDtypeStruct((B, P), jnp.float32),
                  jax.ShapeDtypeStruct((B, P), jnp.int32)),
        mesh=mesh,
        scratch_types=[pltpu.VMEM((4 * P,), jnp.float32),
                       pltpu.VMEM((128,), jnp.float32),
                       pltpu.VMEM((P,), jnp.float32),
                       pltpu.VMEM((P,), jnp.int32)],
        compiler_params=cp,
    )(pri_flat, tgt_pad)

    def prep(x):
        xt = jnp.transpose(x, (0, 2, 1))
        xt = jnp.pad(xt, ((0, 0), (0, 0), (0, PADP)))
        return xt.reshape(B, x.shape[2], ROWS, LANES)

    locT = prep(loc_data)
    confT = prep(conf_data)
    landmT = prep(landm_data)
    prT = jnp.pad(jnp.transpose(priors), ((0, 0), (0, PADP)),
                  constant_values=1.0)
    pxp, pyp, pwp, php = prT[0], prT[1], prT[2], prT[3]
    priT = jnp.stack([pxp, pyp, 1.0 / (VAR0 * pwp), 1.0 / (VAR0 * php),
                      jnp.log(pwp), jnp.log(php)]).reshape(6, ROWS, LANES)
    ctP = jnp.pad(conf_t, ((0, 0), (0, PADP))).reshape(B, 1, ROWS, LANES)
    biP = jnp.pad(btiA, ((0, 0), (0, PADP))).reshape(B, 1, ROWS, LANES)

    o = pl.pallas_call(
        _tc_body,
        grid=(B,),
        in_specs=[
            pl.BlockSpec((1, 4, ROWS, LANES), lambda b: (b, 0, 0, 0)),
            pl.BlockSpec((1, NUM_CLASSES, ROWS, LANES), lambda b: (b, 0, 0, 0)),
            pl.BlockSpec((1, 10, ROWS, LANES), lambda b: (b, 0, 0, 0)),
            pl.BlockSpec((6, ROWS, LANES), lambda b: (0, 0, 0)),
            pl.BlockSpec((1, 8, 15), lambda b: (b, 0, 0)),
            pl.BlockSpec((1, 1, ROWS, LANES), lambda b: (b, 0, 0, 0)),
            pl.BlockSpec((1, 1, ROWS, LANES), lambda b: (b, 0, 0, 0)),
        ],
        out_specs=pl.BlockSpec((1, 1, LANES), lambda b: (b, 0, 0)),
        out_shape=jax.ShapeDtypeStruct((B, 1, LANES), jnp.float32),
    )(locT, confT, landmT, priT, targets, ctP, biP)

    N = jnp.maximum(jnp.sum(o[:, 0, 3]), 1.0)
    N1 = jnp.maximum(jnp.sum(o[:, 0, 4]), 1.0)
    return (jnp.sum(o[:, 0, 0]) / N, jnp.sum(o[:, 0, 1]) / N,
            jnp.sum(o[:, 0, 2]) / N1)


# speedup vs baseline: 86.8786x; 1.4265x over previous
import dataclasses

import jax
import jax.numpy as jnp
from jax.experimental import pallas as pl
from jax.experimental.pallas import tpu as pltpu
from jax.experimental.pallas import tpu_sc as plsc

P = 16800
NUM_CLASSES = 2
THRESHOLD = 0.35
NEGPOS_RATIO = 7
VAR0, VAR1 = 0.1, 0.2
LANES = 128
ROWS = (P + LANES - 1) // LANES
PADP = ROWS * LANES - P
NV = P // 16
NSUB = 16


def _sc_match_body(pri_hbm, tgt_hbm, conf_out, bti_out, pvec, tsm, bto_ref, bti_ref):
    f32 = jnp.float32
    c = jax.lax.axis_index("c")
    s = jax.lax.axis_index("s")
    b = c * NSUB + s
    pltpu.sync_copy(pri_hbm, pvec)
    pltpu.sync_copy(tgt_hbm.at[b], tsm)

    lane = jax.lax.iota(jnp.int32, 16)
    BIG = jnp.int32(P + 2)

    trow = [tsm[pl.ds(16 * i, 16)] for i in range(8)]
    tx1 = [trow[i][0] for i in range(8)]
    ty1 = [trow[i][1] for i in range(8)]
    tx2 = [trow[i][2] for i in range(8)]
    ty2 = [trow[i][3] for i in range(8)]
    lab = [trow[i][14] for i in range(8)]
    at = [(tx2[i] - tx1[i]) * (ty2[i] - ty1[i]) for i in range(8)]

    init = tuple([jnp.full((16,), -2.0, f32) for _ in range(8)]
                 + [jnp.zeros((16,), jnp.int32) for _ in range(8)])

    def body(i, carry):
        rm = list(carry[0:8])
        ra = list(carry[8:16])
        off = i * 16
        gidx = off + lane
        px = pvec[pl.ds(off, 16)]
        py = pvec[pl.ds(P + off, 16)]
        pw = pvec[pl.ds(2 * P + off, 16)]
        ph = pvec[pl.ds(3 * P + off, 16)]
        pf_x1 = px - pw * 0.5
        pf_y1 = py - ph * 0.5
        pf_x2 = px + pw * 0.5
        pf_y2 = py + ph * 0.5
        area_p = (pf_x2 - pf_x1) * (pf_y2 - pf_y1)
        bto_v = jnp.full((16,), -1.0, f32)
        bti_v = jnp.zeros((16,), jnp.int32)
        for i_t in range(8):
            iw = jnp.maximum(
                jnp.minimum(pf_x2, tx2[i_t]) - jnp.maximum(pf_x1, tx1[i_t]), 0.0)
            ih = jnp.maximum(
                jnp.minimum(pf_y2, ty2[i_t]) - jnp.maximum(pf_y1, ty1[i_t]), 0.0)
            inter = iw * ih
            iou = inter / (at[i_t] + area_p - inter)
            upd = iou > bto_v
            bti_v = jnp.where(upd, i_t, bti_v)
            bto_v = jnp.where(upd, iou, bto_v)
            upd2 = iou > rm[i_t]
            ra[i_t] = jnp.where(upd2, gidx, ra[i_t])
            rm[i_t] = jnp.where(upd2, iou, rm[i_t])
        bto_ref[pl.ds(off, 16)] = bto_v
        bti_ref[pl.ds(off, 16)] = bti_v
        return tuple(rm + ra)

    carry = jax.lax.fori_loop(0, NV, body, init)
    rm = carry[0:8]
    ra = carry[8:16]

    m = [jnp.max(rm[i]) for i in range(8)]
    bpi = [jnp.min(jnp.where(rm[i] == m[i], ra[i], BIG)) for i in range(8)]
    base = [jnp.bitwise_and(bpi[i], jnp.int32(~15)) for i in range(8)]
    hit = [(base[i] + lane) == bpi[i] for i in range(8)]
    orig = [jnp.sum(jnp.where(hit[i], bto_ref[pl.ds(base[i], 16)], 0.0))
            for i in range(8)]
    for i in range(8):
        fill = jnp.where(m[i] >= 0.2, 2.0, orig[i])
        sl = bto_ref[pl.ds(base[i], 16)]
        bto_ref[pl.ds(base[i], 16)] = jnp.where(hit[i], fill, sl)
        sli = bti_ref[pl.ds(base[i], 16)]
        bti_ref[pl.ds(base[i], 16)] = jnp.where(hit[i], jnp.int32(i), sli)

    def conf_body(i, _):
        off = i * 16
        bto_v = bto_ref[pl.ds(off, 16)]
        bti_v = bti_ref[pl.ds(off, 16)]
        cv = jnp.zeros((16,), f32)
        for i_t in range(8):
            cv = jnp.where(bti_v == i_t, lab[i_t], cv)
        cv = jnp.where(bto_v < THRESHOLD, 0.0, cv)
        bto_ref[pl.ds(off, 16)] = cv
        return 0

    jax.lax.fori_loop(0, NV, conf_body, 0)
    pltpu.sync_copy(bto_ref, conf_out.at[b])
    pltpu.sync_copy(bti_ref, bti_out.at[b])


def _tc_body(loc_ref, conf_ref, landm_ref, pri_ref, tgt_ref, ct_ref, bi_ref,
             out_ref, s_out_ref, ce_out_ref):
    f32 = jnp.float32
    row = jax.lax.broadcasted_iota(jnp.int32, (ROWS, LANES), 0)
    lane = jax.lax.broadcasted_iota(jnp.int32, (ROWS, LANES), 1)
    flat_idx = row * LANES + lane
    valid = flat_idx < P

    px = pri_ref[0]
    py = pri_ref[1]
    inv_vpw = pri_ref[2]
    inv_vph = pri_ref[3]
    log_pw = pri_ref[4]
    log_ph = pri_ref[5]

    conf_t = jnp.where(valid, ct_ref[0], 0.0)
    bti = bi_ref[0]
    pos = conf_t != 0.0
    pos1 = conf_t > 0.0
    posf = pos.astype(f32)
    pos1f = pos1.astype(f32)
    num_pos = jnp.sum(posf)
    num_pos_landm = jnp.sum(pos1f)

    bb0 = (bti & 1) != 0
    bb1 = (bti & 2) != 0
    bb2 = (bti & 4) != 0

    def tree8(vals):
        w = [jnp.where(bb0, vals[2 * j + 1], vals[2 * j]) for j in range(4)]
        x = [jnp.where(bb1, w[2 * k + 1], w[2 * k]) for k in range(2)]
        return jnp.where(bb2, x[1], x[0])

    def sl1(dv):
        a = jnp.abs(dv)
        return jnp.where(a < 1.0, 0.5 * dv * dv, a - 0.5)

    tx1 = [tgt_ref[0, i, 0] for i in range(8)]
    ty1 = [tgt_ref[0, i, 1] for i in range(8)]
    tx2 = [tgt_ref[0, i, 2] for i in range(8)]
    ty2 = [tgt_ref[0, i, 3] for i in range(8)]
    cxm = [(tx1[i] + tx2[i]) * 0.5 for i in range(8)]
    cym = [(ty1[i] + ty2[i]) * 0.5 for i in range(8)]
    logw = [jnp.log(tx2[i] - tx1[i]) for i in range(8)]
    logh = [jnp.log(ty2[i] - ty1[i]) for i in range(8)]

    g_cx = (tree8(cxm) - px) * inv_vpw
    g_cy = (tree8(cym) - py) * inv_vph
    g_w = (tree8(logw) - log_pw) / VAR1
    g_h = (tree8(logh) - log_ph) / VAR1
    loss_l = jnp.sum(
        (sl1(loc_ref[0, 0] - g_cx) + sl1(loc_ref[0, 1] - g_cy)
         + sl1(loc_ref[0, 2] - g_w) + sl1(loc_ref[0, 3] - g_h)) * posf)

    lacc = jnp.zeros((ROWS, LANES), f32)
    for j in range(10):
        mlm = tree8([tgt_ref[0, i, 4 + j] for i in range(8)])
        pc = px if j % 2 == 0 else py
        pi = inv_vpw if j % 2 == 0 else inv_vph
        g = (mlm - pc) * pi
        lacc = lacc + sl1(landm_ref[0, j] - g)
    loss_landm = jnp.sum(lacc * pos1f)

    c0 = conf_ref[0, 0]
    c1 = conf_ref[0, 1]
    d = c1 - c0
    lse = jnp.maximum(c0, c1) + jnp.log(1.0 + jnp.exp(-jnp.abs(d)))
    ce_pos = lse - c1
    ce_neg = lse - c0
    pos_sum = jnp.sum(ce_pos * posf)

    SENT = jnp.int32(-2147483648)
    bits = jax.lax.bitcast_convert_type(d, jnp.int32)
    s = jnp.where(bits >= 0, bits, SENT - bits - 1)
    s = jnp.where(pos | jnp.logical_not(valid), SENT, s)
    s_out_ref[...] = s.reshape(1, ROWS, LANES)
    ce_out_ref[...] = ce_neg.reshape(1, ROWS, LANES)

    lane_row = jax.lax.broadcasted_iota(jnp.int32, (1, LANES), 1)
    out = (jnp.where(lane_row == 0, loss_l, 0.0)
           + jnp.where(lane_row == 1, pos_sum, 0.0)
           + jnp.where(lane_row == 2, loss_landm, 0.0)
           + jnp.where(lane_row == 3, num_pos, 0.0)
           + jnp.where(lane_row == 4, num_pos_landm, 0.0))
    out_ref[...] = out.reshape(1, 1, LANES)


def _mine_body(o1_ref, s_ref, ce_ref, out_ref):
    f32 = jnp.float32
    B = s_ref.shape[0]
    SENT = jnp.int32(-2147483648)
    s = s_ref[...]
    ce = ce_ref[...]
    npos = o1_ref[:, :, 3:4]
    K = jnp.minimum(jnp.int32(NEGPOS_RATIO) * npos.astype(jnp.int32),
                    jnp.int32(P - 1))

    def red(x):
        return jnp.sum(jnp.sum(x, axis=2, keepdims=True), axis=1,
                       keepdims=True)

    def bisect_body(_, carry):
        lo, hi = carry
        mid = (lo >> 1) + (hi >> 1) + ((lo | hi) & jnp.int32(1))
        cnt = red((s >= mid).astype(jnp.int32))
        take = cnt >= K
        return (jnp.where(take, mid, lo), jnp.where(take, hi, mid - jnp.int32(1)))

    lo, _ = jax.lax.fori_loop(
        0, 32, bisect_body,
        (jnp.full((B, 1, 1), SENT, jnp.int32),
         jnp.full((B, 1, 1), 2147483647, jnp.int32)))
    T = lo
    gt = (s > T).astype(f32)
    eq = (s == T).astype(f32)
    c_gt = red(gt)
    c_eq = red(eq)
    k_rem = jnp.where(T > SENT, K.astype(f32) - c_gt, 0.0)
    neg = red(ce * gt) + k_rem / jnp.maximum(c_eq, 1.0) * red(ce * eq)
    lane_row = jax.lax.broadcasted_iota(jnp.int32, (B, 1, LANES), 2)
    out_ref[...] = jnp.where(lane_row == 0, neg, 0.0)


def kernel(loc_data, conf_data, landm_data, priors, targets):
    B = loc_data.shape[0]
    targets = jax.lax.stop_gradient(targets)
    priors = jax.lax.stop_gradient(priors)

    pri_flat = jnp.transpose(priors).reshape(4 * P)
    tgt_pad = jnp.pad(targets, ((0, 0), (0, 0), (0, 1))).reshape(B, 128)
    mesh = plsc.VectorSubcoreMesh(core_axis_name="c", subcore_axis_name="s")
    cp = pltpu.CompilerParams()
    if "needs_layout_passes" in pltpu.CompilerParams.__dataclass_fields__:
        cp = dataclasses.replace(cp, needs_layout_passes=False)
    conf_t, btiA = pl.kernel(
        _sc_match_body,
        out_type=(jax.ShapeDtypeStruct((B, P), jnp.float32),
                  jax.ShapeDtypeStruct((B, P), jnp.int32)),
        mesh=mesh,
        scratch_types=[pltpu.VMEM((4 * P,), jnp.float32),
                       pltpu.VMEM((128,), jnp.float32),
                       pltpu.VMEM((P,), jnp.float32),
                       pltpu.VMEM((P,), jnp.int32)],
        compiler_params=cp,
    )(pri_flat, tgt_pad)

    def prep(x):
        xt = jnp.transpose(x, (0, 2, 1))
        xt = jnp.pad(xt, ((0, 0), (0, 0), (0, PADP)))
        return xt.reshape(B, x.shape[2], ROWS, LANES)

    locT = prep(loc_data)
    confT = prep(conf_data)
    landmT = prep(landm_data)
    prT = jnp.pad(jnp.transpose(priors), ((0, 0), (0, PADP)),
                  constant_values=1.0)
    pxp, pyp, pwp, php = prT[0], prT[1], prT[2], prT[3]
    priT = jnp.stack([pxp, pyp, 1.0 / (VAR0 * pwp), 1.0 / (VAR0 * php),
                      jnp.log(pwp), jnp.log(php)]).reshape(6, ROWS, LANES)
    ctP = jnp.pad(conf_t, ((0, 0), (0, PADP))).reshape(B, 1, ROWS, LANES)
    biP = jnp.pad(btiA, ((0, 0), (0, PADP))).reshape(B, 1, ROWS, LANES)

    o = pl.pallas_call(
        _tc_body,
        grid=(B,),
        in_specs=[
            pl.BlockSpec((1, 4, ROWS, LANES), lambda b: (b, 0, 0, 0)),
            pl.BlockSpec((1, NUM_CLASSES, ROWS, LANES), lambda b: (b, 0, 0, 0)),
            pl.BlockSpec((1, 10, ROWS, LANES), lambda b: (b, 0, 0, 0)),
            pl.BlockSpec((6, ROWS, LANES), lambda b: (0, 0, 0)),
            pl.BlockSpec((1, 8, 15), lambda b: (b, 0, 0)),
            pl.BlockSpec((1, 1, ROWS, LANES), lambda b: (b, 0, 0, 0)),
            pl.BlockSpec((1, 1, ROWS, LANES), lambda b: (b, 0, 0, 0)),
        ],
        out_specs=[
            pl.BlockSpec((1, 1, LANES), lambda b: (b, 0, 0)),
            pl.BlockSpec((1, ROWS, LANES), lambda b: (b, 0, 0)),
            pl.BlockSpec((1, ROWS, LANES), lambda b: (b, 0, 0)),
        ],
        out_shape=[
            jax.ShapeDtypeStruct((B, 1, LANES), jnp.float32),
            jax.ShapeDtypeStruct((B, ROWS, LANES), jnp.int32),
            jax.ShapeDtypeStruct((B, ROWS, LANES), jnp.float32),
        ],
    )(locT, confT, landmT, priT, targets, ctP, biP)
    o, sA, ceA = o

    o2 = pl.pallas_call(
        _mine_body,
        out_shape=jax.ShapeDtypeStruct((B, 1, LANES), jnp.float32),
    )(o, sA, ceA)

    N = jnp.maximum(jnp.sum(o[:, 0, 3]), 1.0)
    N1 = jnp.maximum(jnp.sum(o[:, 0, 4]), 1.0)
    lc = jnp.sum(o[:, 0, 1]) + jnp.sum(o2[:, 0, 0])
    return (jnp.sum(o[:, 0, 0]) / N, lc / N,
            jnp.sum(o[:, 0, 2]) / N1)
